# baseline (speedup 1.0000x reference)
import functools

import jax
import jax.numpy as jnp
from jax import lax
from jax.experimental import pallas as pl
from jax.experimental.pallas import tpu as pltpu
from jax.experimental.pallas import tpu_sc as plsc

NC = 2
NS = 16
NW = NC * NS
L = 16
CHUNK = 128


NSLOT = 8


def _agg_body(nchunk, rows_z, table, srcp, dstp, zeros_hbm, out,
              srcv, dstv, rows, acc, gsem, ssem):
  cid = lax.axis_index("c")
  sid = lax.axis_index("s")
  wid = cid * NS + sid

  pltpu.sync_copy(srcp.at[wid], srcv)
  pltpu.sync_copy(dstp.at[wid], dstv)

  pltpu.sync_copy(zeros_hbm, acc.at[pl.ds(sid * rows_z, rows_z)])
  plsc.subcore_barrier()

  for s in range(NSLOT):
    pltpu.async_copy(table.at[srcv.at[s]], rows.at[s], gsem[s])

  def body(i, carry):
    c0 = NSLOT * i
    for s in range(NSLOT):
      pltpu.make_async_copy(table.at[srcv.at[c0 + s]], rows.at[s],
                            gsem[s]).wait()
      pltpu.async_copy(rows.at[s], acc.at[dstv.at[c0 + s]], ssem[s],
                       add=True)
    for s in range(NSLOT):
      pltpu.make_async_copy(rows.at[s], acc.at[dstv.at[0]], ssem[s]).wait()
      pltpu.async_copy(table.at[srcv.at[c0 + NSLOT + s]], rows.at[s], gsem[s])
    return carry
  lax.fori_loop(0, nchunk // NSLOT, body, 0)

  for s in range(NSLOT):
    pltpu.make_async_copy(table.at[srcv.at[nchunk + s]], rows.at[s],
                          gsem[s]).wait()

  plsc.subcore_barrier()
  pltpu.sync_copy(acc.at[pl.ds(sid * rows_z, rows_z)],
                  out.at[cid, pl.ds(sid * rows_z, rows_z)])


def _make_agg(n_acc, nchunk, rows_z):
  return pl.kernel(
      functools.partial(_agg_body, nchunk, rows_z),
      out_type=jax.ShapeDtypeStruct((NC, n_acc, L), jnp.float32),
      mesh=plsc.VectorSubcoreMesh(core_axis_name="c", subcore_axis_name="s"),
      compiler_params=pltpu.CompilerParams(use_tc_tiling_on_sc=False),
      scratch_types=[
          pltpu.VMEM((nchunk + NSLOT, CHUNK), jnp.int32),
          pltpu.VMEM((nchunk + NSLOT, CHUNK), jnp.int32),
          pltpu.VMEM((NSLOT, CHUNK, L), jnp.float32),
          pltpu.VMEM_SHARED((n_acc, L), jnp.float32),
          [pltpu.SemaphoreType.DMA] * NSLOT,
          [pltpu.SemaphoreType.DMA] * NSLOT,
      ],
  )


def _proj_body(x_ref, w_ref, y_ref):
  y_ref[...] = jnp.dot(x_ref[...], w_ref[...],
                       precision=jax.lax.Precision.HIGHEST,
                       preferred_element_type=jnp.float32)


def _mlp1_body(eps_ref, y_ref, a0_ref, a1_ref, b1_ref, w2_ref, b2_ref,
               w3_ref, z_ref):
  e = 1.0 + eps_ref[0]
  hp = e * y_ref[...] + a0_ref[...] + a1_ref[...] + b1_ref[...]
  h = jnp.maximum(hp, 0.0)
  x1 = jnp.maximum(
      jnp.dot(h, w2_ref[...], precision=jax.lax.Precision.HIGHEST,
              preferred_element_type=jnp.float32)
      + b2_ref[...], 0.0)
  z_ref[...] = jnp.dot(x1, w3_ref[...], precision=jax.lax.Precision.HIGHEST,
                       preferred_element_type=jnp.float32)


def _mlp2_body(eps_ref, z_ref, a0_ref, a1_ref, b3_ref, w4_ref, b4_ref,
               o_ref):
  e = 1.0 + eps_ref[0]
  hp = e * z_ref[...] + a0_ref[...] + a1_ref[...] + b3_ref[...]
  g = jnp.maximum(hp, 0.0)
  o = jnp.dot(g, w4_ref[...], precision=jax.lax.Precision.HIGHEST,
              preferred_element_type=jnp.float32) + b4_ref[...]
  m = jnp.max(o, axis=1, keepdims=True)
  s = o - m
  o_ref[...] = s - jnp.log(jnp.sum(jnp.exp(s), axis=1, keepdims=True))


def _smem_first_specs(n):
  return [pl.BlockSpec(memory_space=pltpu.SMEM)] + \
         [pl.BlockSpec(memory_space=pltpu.VMEM)] * (n - 1)


@jax.jit
def kernel(x, edge_index, eps1, W1, b1, W2, b2, eps2, W3, b3, W4, b4):
  n, f_in = x.shape
  e = edge_index.shape[1]
  h = W1.shape[1]

  nchunk = -(-e // (NW * CHUNK * NSLOT)) * NSLOT
  ept = nchunk * CHUNK
  e_pad = NW * ept
  rows_z = -(-(n + 1) // (NS * 8)) * 8
  n_acc = NS * rows_z

  src = edge_index[0]
  dst = edge_index[1]
  pad = e_pad - e
  extra = jnp.zeros((NW, NSLOT, CHUNK), jnp.int32)
  srcp = jnp.concatenate([
      jnp.concatenate([src, jnp.zeros((pad,), jnp.int32)]).reshape(
          NW, nchunk, CHUNK), extra], axis=1)
  dstp = jnp.concatenate([
      jnp.concatenate([dst, jnp.full((pad,), n, jnp.int32)]).reshape(
          NW, nchunk, CHUNK), extra + n], axis=1)
  zrows = jnp.zeros((rows_z, L), jnp.float32)

  agg = _make_agg(n_acc, nchunk, rows_z)

  y = pl.pallas_call(
      _proj_body,
      out_shape=jax.ShapeDtypeStruct((n, h), jnp.float32),
  )(x, W1)

  ay = agg(y, srcp, dstp, zrows)

  z = pl.pallas_call(
      _mlp1_body,
      out_shape=jax.ShapeDtypeStruct((n, h), jnp.float32),
      in_specs=_smem_first_specs(8),
  )(eps1.reshape(1), y, ay[0, :n], ay[1, :n], b1.reshape(1, h), W2,
    b2.reshape(1, h), W3)

  az = agg(z, srcp, dstp, zrows)

  c = W4.shape[1]
  out = pl.pallas_call(
      _mlp2_body,
      out_shape=jax.ShapeDtypeStruct((n, c), jnp.float32),
      in_specs=_smem_first_specs(7),
  )(eps2.reshape(1), z, az[0, :n], az[1, :n], b3.reshape(1, c), W4,
    b4.reshape(1, c))
  return out


# speedup vs baseline: 1.3616x; 1.3616x over previous
import functools

import jax
import jax.numpy as jnp
from jax import lax
from jax.experimental import pallas as pl
from jax.experimental.pallas import tpu as pltpu
from jax.experimental.pallas import tpu_sc as plsc

NC = 2
NS = 16
NW = NC * NS
L = 16
CHUNK = 128


NSLOT = 4


def _agg_body(nchunk, rows_z, table, srcp, dstp, zeros_hbm, out,
              srcv, dstv, rows, acc, gsem):
  cid = lax.axis_index("c")
  sid = lax.axis_index("s")
  wid = cid * NS + sid

  pltpu.sync_copy(srcp.at[wid], srcv)
  pltpu.sync_copy(dstp.at[wid], dstv)

  pltpu.sync_copy(zeros_hbm, acc.at[pl.ds(sid * rows_z, rows_z)])
  plsc.subcore_barrier()

  for s in range(NSLOT):
    pltpu.async_copy(table.at[srcv.at[s]], rows.at[s], gsem[s])

  def body(i, carry):
    c0 = NSLOT * i
    for s in range(NSLOT):
      pltpu.make_async_copy(table.at[srcv.at[c0 + s]], rows.at[s],
                            gsem[s]).wait()
      pltpu.sync_copy(rows.at[s], acc.at[dstv.at[c0 + s]], add=True)
      pltpu.async_copy(table.at[srcv.at[c0 + NSLOT + s]], rows.at[s], gsem[s])
    return carry
  lax.fori_loop(0, nchunk // NSLOT, body, 0)

  for s in range(NSLOT):
    pltpu.make_async_copy(table.at[srcv.at[nchunk + s]], rows.at[s],
                          gsem[s]).wait()

  plsc.subcore_barrier()
  pltpu.sync_copy(acc.at[pl.ds(sid * rows_z, rows_z)],
                  out.at[cid, pl.ds(sid * rows_z, rows_z)])


def _make_agg(n_acc, nchunk, rows_z):
  return pl.kernel(
      functools.partial(_agg_body, nchunk, rows_z),
      out_type=jax.ShapeDtypeStruct((NC, n_acc, L), jnp.float32),
      mesh=plsc.VectorSubcoreMesh(core_axis_name="c", subcore_axis_name="s"),
      compiler_params=pltpu.CompilerParams(use_tc_tiling_on_sc=False),
      scratch_types=[
          pltpu.VMEM((nchunk + NSLOT, CHUNK), jnp.int32),
          pltpu.VMEM((nchunk + NSLOT, CHUNK), jnp.int32),
          pltpu.VMEM((NSLOT, CHUNK, L), jnp.float32),
          pltpu.VMEM_SHARED((n_acc, L), jnp.float32),
          [pltpu.SemaphoreType.DMA] * NSLOT,
      ],
  )


def _proj_body(x_ref, w_ref, y_ref):
  y_ref[...] = jnp.dot(x_ref[...], w_ref[...],
                       precision=jax.lax.Precision.HIGHEST,
                       preferred_element_type=jnp.float32)


def _mlp1_body(eps_ref, y_ref, a0_ref, a1_ref, b1_ref, w2_ref, b2_ref,
               w3_ref, z_ref):
  e = 1.0 + eps_ref[0]
  hp = e * y_ref[...] + a0_ref[...] + a1_ref[...] + b1_ref[...]
  h = jnp.maximum(hp, 0.0)
  x1 = jnp.maximum(
      jnp.dot(h, w2_ref[...], precision=jax.lax.Precision.HIGHEST,
              preferred_element_type=jnp.float32)
      + b2_ref[...], 0.0)
  z_ref[...] = jnp.dot(x1, w3_ref[...], precision=jax.lax.Precision.HIGHEST,
                       preferred_element_type=jnp.float32)


def _mlp2_body(eps_ref, z_ref, a0_ref, a1_ref, b3_ref, w4_ref, b4_ref,
               o_ref):
  e = 1.0 + eps_ref[0]
  hp = e * z_ref[...] + a0_ref[...] + a1_ref[...] + b3_ref[...]
  g = jnp.maximum(hp, 0.0)
  o = jnp.dot(g, w4_ref[...], precision=jax.lax.Precision.HIGHEST,
              preferred_element_type=jnp.float32) + b4_ref[...]
  m = jnp.max(o, axis=1, keepdims=True)
  s = o - m
  o_ref[...] = s - jnp.log(jnp.sum(jnp.exp(s), axis=1, keepdims=True))


def _smem_first_specs(n):
  return [pl.BlockSpec(memory_space=pltpu.SMEM)] + \
         [pl.BlockSpec(memory_space=pltpu.VMEM)] * (n - 1)


@jax.jit
def kernel(x, edge_index, eps1, W1, b1, W2, b2, eps2, W3, b3, W4, b4):
  n, f_in = x.shape
  e = edge_index.shape[1]
  h = W1.shape[1]

  nchunk = -(-e // (NW * CHUNK * NSLOT)) * NSLOT
  ept = nchunk * CHUNK
  e_pad = NW * ept
  rows_z = -(-(n + 1) // (NS * 8)) * 8
  n_acc = NS * rows_z

  src = edge_index[0]
  dst = edge_index[1]
  pad = e_pad - e
  extra = jnp.zeros((NW, NSLOT, CHUNK), jnp.int32)
  srcp = jnp.concatenate([
      jnp.concatenate([src, jnp.zeros((pad,), jnp.int32)]).reshape(
          NW, nchunk, CHUNK), extra], axis=1)
  dstp = jnp.concatenate([
      jnp.concatenate([dst, jnp.full((pad,), n, jnp.int32)]).reshape(
          NW, nchunk, CHUNK), extra + n], axis=1)
  zrows = jnp.zeros((rows_z, L), jnp.float32)

  agg = _make_agg(n_acc, nchunk, rows_z)

  y = pl.pallas_call(
      _proj_body,
      out_shape=jax.ShapeDtypeStruct((n, h), jnp.float32),
  )(x, W1)

  ay = agg(y, srcp, dstp, zrows)

  z = pl.pallas_call(
      _mlp1_body,
      out_shape=jax.ShapeDtypeStruct((n, h), jnp.float32),
      in_specs=_smem_first_specs(8),
  )(eps1.reshape(1), y, ay[0, :n], ay[1, :n], b1.reshape(1, h), W2,
    b2.reshape(1, h), W3)

  az = agg(z, srcp, dstp, zrows)

  c = W4.shape[1]
  out = pl.pallas_call(
      _mlp2_body,
      out_shape=jax.ShapeDtypeStruct((n, c), jnp.float32),
      in_specs=_smem_first_specs(7),
  )(eps2.reshape(1), z, az[0, :n], az[1, :n], b3.reshape(1, c), W4,
    b4.reshape(1, c))
  return out


# speedup vs baseline: 1.6898x; 1.2410x over previous
import functools

import jax
import jax.numpy as jnp
from jax import lax
from jax.experimental import pallas as pl
from jax.experimental.pallas import tpu as pltpu
from jax.experimental.pallas import tpu_sc as plsc

NC = 2
NS = 16
NW = NC * NS
L = 16
CHUNK = 128


NSLOT = 2


def _agg_body(nchunk, rows_z, table, srcp, dstp, zeros_hbm, out,
              srcv, dstv, rows0, rows1, acc, sem0, sem1):
  cid = lax.axis_index("c")
  sid = lax.axis_index("s")
  wid = cid * NS + sid

  pltpu.sync_copy(srcp.at[wid], srcv)
  pltpu.sync_copy(dstp.at[wid], dstv)

  pltpu.sync_copy(zeros_hbm, acc.at[pl.ds(sid * rows_z, rows_z)])
  plsc.subcore_barrier()

  pltpu.async_copy(table.at[srcv.at[0]], rows0, sem0)
  pltpu.async_copy(table.at[srcv.at[1]], rows1, sem1)

  def body(c2, carry):
    c = 2 * c2
    pltpu.make_async_copy(table.at[srcv.at[c]], rows0, sem0).wait()
    pltpu.sync_copy(rows0, acc.at[dstv.at[c]], add=True)
    pltpu.async_copy(table.at[srcv.at[c + 2]], rows0, sem0)
    pltpu.make_async_copy(table.at[srcv.at[c + 1]], rows1, sem1).wait()
    pltpu.sync_copy(rows1, acc.at[dstv.at[c + 1]], add=True)
    pltpu.async_copy(table.at[srcv.at[c + 3]], rows1, sem1)
    return carry
  lax.fori_loop(0, nchunk // 2, body, 0)

  pltpu.make_async_copy(table.at[srcv.at[nchunk]], rows0, sem0).wait()
  pltpu.make_async_copy(table.at[srcv.at[nchunk + 1]], rows1, sem1).wait()

  plsc.subcore_barrier()
  pltpu.sync_copy(acc.at[pl.ds(sid * rows_z, rows_z)],
                  out.at[cid, pl.ds(sid * rows_z, rows_z)])


def _make_agg(n_acc, nchunk, rows_z):
  return pl.kernel(
      functools.partial(_agg_body, nchunk, rows_z),
      out_type=jax.ShapeDtypeStruct((NC, n_acc, L), jnp.float32),
      mesh=plsc.VectorSubcoreMesh(core_axis_name="c", subcore_axis_name="s"),
      compiler_params=pltpu.CompilerParams(use_tc_tiling_on_sc=False),
      scratch_types=[
          pltpu.VMEM((nchunk + NSLOT, CHUNK), jnp.int32),
          pltpu.VMEM((nchunk + NSLOT, CHUNK), jnp.int32),
          pltpu.VMEM((CHUNK, L), jnp.float32),
          pltpu.VMEM((CHUNK, L), jnp.float32),
          pltpu.VMEM_SHARED((n_acc, L), jnp.float32),
          pltpu.SemaphoreType.DMA,
          pltpu.SemaphoreType.DMA,
      ],
  )


def _proj_body(x_ref, w_ref, y_ref):
  y_ref[...] = jnp.dot(x_ref[...], w_ref[...],
                       precision=jax.lax.Precision.HIGHEST,
                       preferred_element_type=jnp.float32)


def _mlp1_body(eps_ref, y_ref, a0_ref, a1_ref, b1_ref, w2_ref, b2_ref,
               w3_ref, z_ref):
  e = 1.0 + eps_ref[0]
  hp = e * y_ref[...] + a0_ref[...] + a1_ref[...] + b1_ref[...]
  h = jnp.maximum(hp, 0.0)
  x1 = jnp.maximum(
      jnp.dot(h, w2_ref[...], precision=jax.lax.Precision.HIGHEST,
              preferred_element_type=jnp.float32)
      + b2_ref[...], 0.0)
  z_ref[...] = jnp.dot(x1, w3_ref[...], precision=jax.lax.Precision.HIGHEST,
                       preferred_element_type=jnp.float32)


def _mlp2_body(eps_ref, z_ref, a0_ref, a1_ref, b3_ref, w4_ref, b4_ref,
               o_ref):
  e = 1.0 + eps_ref[0]
  hp = e * z_ref[...] + a0_ref[...] + a1_ref[...] + b3_ref[...]
  g = jnp.maximum(hp, 0.0)
  o = jnp.dot(g, w4_ref[...], precision=jax.lax.Precision.HIGHEST,
              preferred_element_type=jnp.float32) + b4_ref[...]
  m = jnp.max(o, axis=1, keepdims=True)
  s = o - m
  o_ref[...] = s - jnp.log(jnp.sum(jnp.exp(s), axis=1, keepdims=True))


def _smem_first_specs(n):
  return [pl.BlockSpec(memory_space=pltpu.SMEM)] + \
         [pl.BlockSpec(memory_space=pltpu.VMEM)] * (n - 1)


@jax.jit
def kernel(x, edge_index, eps1, W1, b1, W2, b2, eps2, W3, b3, W4, b4):
  n, f_in = x.shape
  e = edge_index.shape[1]
  h = W1.shape[1]

  nchunk = -(-e // (NW * CHUNK * NSLOT)) * NSLOT
  ept = nchunk * CHUNK
  e_pad = NW * ept
  rows_z = -(-(n + 1) // (NS * 8)) * 8
  n_acc = NS * rows_z

  src = edge_index[0]
  dst = edge_index[1]
  pad = e_pad - e
  extra = jnp.zeros((NW, NSLOT, CHUNK), jnp.int32)
  srcp = jnp.concatenate([
      jnp.concatenate([src, jnp.zeros((pad,), jnp.int32)]).reshape(
          NW, nchunk, CHUNK), extra], axis=1)
  dstp = jnp.concatenate([
      jnp.concatenate([dst, jnp.full((pad,), n, jnp.int32)]).reshape(
          NW, nchunk, CHUNK), extra + n], axis=1)
  zrows = jnp.zeros((rows_z, L), jnp.float32)

  agg = _make_agg(n_acc, nchunk, rows_z)

  y = pl.pallas_call(
      _proj_body,
      out_shape=jax.ShapeDtypeStruct((n, h), jnp.float32),
  )(x, W1)

  ay = agg(y, srcp, dstp, zrows)

  z = pl.pallas_call(
      _mlp1_body,
      out_shape=jax.ShapeDtypeStruct((n, h), jnp.float32),
      in_specs=_smem_first_specs(8),
  )(eps1.reshape(1), y, ay[0, :n], ay[1, :n], b1.reshape(1, h), W2,
    b2.reshape(1, h), W3)

  az = agg(z, srcp, dstp, zrows)

  c = W4.shape[1]
  out = pl.pallas_call(
      _mlp2_body,
      out_shape=jax.ShapeDtypeStruct((n, c), jnp.float32),
      in_specs=_smem_first_specs(7),
  )(eps2.reshape(1), z, az[0, :n], az[1, :n], b3.reshape(1, c), W4,
    b4.reshape(1, c))
  return out
